# Initial kernel scaffold; baseline (speedup 1.0000x reference)
import jax
import jax.numpy as jnp
from jax.experimental import pallas as pl


def kernel(x, conv_w, bn_gamma, bn_beta, bn_mean, bn_var):
    raise NotImplementedError("write your pallas kernel here")


# speedup vs baseline: 1.9587x; 1.9587x over previous
import jax
import jax.numpy as jnp
from jax.experimental import pallas as pl
from jax.experimental.pallas import tpu as pltpu

_K = 7
_PAD = (_K - 1) // 2


def _sam_kernel(x_ref, m_ref, shift_ref, o_ref, pooled_ref):
    Bt, C, HW = x_ref.shape
    pool_rows = pooled_ref.shape[0]

    if pool_rows > Bt:
        pooled_ref[Bt:pool_rows, :] = jnp.zeros(
            (pool_rows - Bt, 2 * HW), jnp.float32)

    if C % 32 == 0:
        ch = 32
    elif C % 16 == 0:
        ch = 16
    elif C % 8 == 0:
        ch = 8
    else:
        ch = 1
    n_chunks = C // ch

    for b in range(Bt):
        if ch >= 8:

            def pool_body(i, carry, b=b):
                am, asm = carry
                c0 = pl.multiple_of(i * ch, ch)
                blk = x_ref[b, pl.ds(c0, ch), :]
                blk3 = blk.reshape(ch // 8, 8, HW)
                am = jnp.maximum(am, jnp.max(blk3, axis=0))
                asm = asm + jnp.sum(blk3.astype(jnp.float32), axis=0)
                return am, asm

            am, asm = jax.lax.fori_loop(
                0, n_chunks, pool_body,
                (jnp.full((8, HW), -jnp.inf, dtype=x_ref.dtype),
                 jnp.zeros((8, HW), jnp.float32)),
                unroll=2)
            p_max = jnp.max(am, axis=0, keepdims=True)
            p_sum = jnp.sum(asm, axis=0, keepdims=True)
        else:
            p_max = x_ref[b, 0:1, :]
            p_sum = p_max.astype(jnp.float32)
            for c in range(1, C):
                xc = x_ref[b, c:c + 1, :]
                p_max = jnp.maximum(p_max, xc)
                p_sum = p_sum + xc.astype(jnp.float32)

        pooled_ref[b:b + 1, 0:HW] = p_max.astype(jnp.float32)
        pooled_ref[b:b + 1, HW:2 * HW] = p_sum

    conv = jnp.dot(pooled_ref[...], m_ref[...],
                   precision=jax.lax.Precision.HIGHEST,
                   preferred_element_type=jnp.float32)
    gate = jax.nn.sigmoid(conv + shift_ref[0])
    if o_ref.dtype == jnp.bfloat16:
        gate = gate.astype(jnp.bfloat16)

    for b in range(Bt):
        g = gate[b:b + 1, :]
        if ch >= 8:

            def gate_body(i, carry, b=b, g=g):
                c0 = pl.multiple_of(i * ch, ch)
                xblk = x_ref[b, pl.ds(c0, ch), :]
                o_ref[b, pl.ds(c0, ch), :] = (xblk * g).astype(o_ref.dtype)
                return carry

            jax.lax.fori_loop(0, n_chunks, gate_body, 0, unroll=2)
        else:
            o_ref[b] = (x_ref[b] * g).astype(o_ref.dtype)


def _build_conv_matrix(w_folded, H, W):
    hi = jnp.arange(H)
    relh = hi[:, None] - hi[None, :] + _PAD
    sy = (relh[None] == jnp.arange(_K)[:, None, None])
    sy = sy.astype(jnp.float32)

    wi = jnp.arange(W)
    relw = wi[:, None] - wi[None, :] + _PAD
    validw = (relw >= 0) & (relw < _K)
    relwc = jnp.where(validw, relw, 0)
    tx = jnp.where(validw[None, None],
                   w_folded[:, :, relwc], 0.0)

    m = jnp.einsum("dhH,cdwW->chwHW", sy, tx,
                   precision=jax.lax.Precision.HIGHEST)
    return m.reshape(2 * H * W, H * W)


def _pick_batch_tile(B, bytes_per_elem, target_bytes=4 * 1024 * 1024, max_bt=8):
    bt = max(1, min(B, max_bt, target_bytes // max(bytes_per_elem, 1)))
    while bt > 1 and B // bt < 2:
        bt -= 1
    while B % bt:
        bt -= 1
    return bt


def kernel(x, conv_w, bn_gamma, bn_beta, bn_mean, bn_var, eps=1e-5):
    B, C, H, W = x.shape
    HW = H * W

    bn_scale = bn_gamma / jnp.sqrt(bn_var + eps)
    bn_shift = bn_beta - bn_mean * bn_scale

    w = conv_w.reshape(2, _K, _K).astype(jnp.float32)
    w_folded = jnp.stack([w[0] * bn_scale, w[1] * (bn_scale / C)])

    conv_mat = _build_conv_matrix(w_folded, H, W)
    shift_arr = jnp.reshape(bn_shift, (1,)).astype(jnp.float32)

    x_flat = x.reshape(B, C, HW)
    bt = _pick_batch_tile(B, C * HW * x.dtype.itemsize)
    grid = (B // bt,)
    pool_rows = ((bt + 7) // 8) * 8

    vmem_limit = 48 * 1024 * 1024

    out_flat = pl.pallas_call(
        _sam_kernel,
        out_shape=jax.ShapeDtypeStruct((B, C, HW), x.dtype),
        grid=grid,
        in_specs=[
            pl.BlockSpec((bt, C, HW), lambda i: (i, 0, 0)),
            pl.BlockSpec((2 * HW, HW), lambda i: (0, 0)),
            pl.BlockSpec(memory_space=pltpu.MemorySpace.SMEM),
        ],
        out_specs=pl.BlockSpec((bt, C, HW), lambda i: (i, 0, 0)),
        scratch_shapes=[pltpu.VMEM((pool_rows, 2 * HW), jnp.float32)],
        compiler_params=pltpu.CompilerParams(
            dimension_semantics=("parallel",),
            vmem_limit_bytes=vmem_limit,
        ),
    )(x_flat, conv_mat, shift_arr)

    return out_flat.reshape(B, C, H, W)
